# Initial kernel scaffold; baseline (speedup 1.0000x reference)
import functools

import jax
import jax.numpy as jnp
from jax import lax
from jax.experimental import pallas as pl
from jax.experimental.pallas import tpu as pltpu
from jax.experimental.pallas import tpu_sc as plsc

N = 100000
E = 1600000
G = 512
D_IN = 9
DH = 64

NC = 2
NS = 16
LANES = 16

NBKT = 4
BK = N // NBKT
ACC_ROWS = 25088
SUB_ROWS = ACC_ROWS // NS
CE = 400
EPS = E // NS
NCH = EPS // CE

_mesh = plsc.VectorSubcoreMesh(core_axis_name="c", subcore_axis_name="s")


def _zero_acc(acc, zbuf, s):
    lo = s * SUB_ROWS

    @pl.loop(0, SUB_ROWS // 256)
    def _(k):
        pltpu.sync_copy(zbuf, acc.at[pl.ds(lo + k * 256, 256)])
    rem = SUB_ROWS % 256
    if rem:
        pltpu.sync_copy(zbuf.at[pl.ds(0, rem)],
                        acc.at[pl.ds(lo + (SUB_ROWS // 256) * 256, rem)])


def _fill(buf, value):
    v = jnp.full((LANES,), value, jnp.float32)

    @pl.loop(0, buf.shape[0])
    def _(r):
        for q in range(DH // LANES):
            buf[r, pl.ds(q * LANES, LANES)] = v


def _writeback(acc, out_hbm, bkt, s):
    lo = jnp.minimum(s * SUB_ROWS, BK - SUB_ROWS)
    pltpu.sync_copy(acc.at[pl.ds(lo, SUB_ROWS)],
                    out_hbm.at[pl.ds(bkt * BK + lo, SUB_ROWS)])


def _mask_dloc(didx, dloc, lo):
    @pl.loop(0, CE // LANES)
    def _(i):
        dv = didx[pl.ds(i * LANES, LANES)]
        inb = (dv >= lo) & (dv < lo + BK)
        dloc[pl.ds(i * LANES, LANES)] = jnp.where(inb, dv - lo, BK + (dv & 63))


def _sc_common(src_hbm, dst_hbm, g_hbm, out_hbm, sidx, didx, dloc, gbuf,
               zbuf, acc, *, gather):
    c = lax.axis_index("c")
    s = lax.axis_index("s")
    if not gather:
        _fill(gbuf, 1.0)
    _fill(zbuf, 0.0)
    for p in range(NC):
        bkt = NC * c + p
        lo = bkt * BK
        _zero_acc(acc, zbuf, s)
        plsc.subcore_barrier()

        @pl.loop(0, NCH)
        def _(ch):
            base = s * EPS + ch * CE
            pltpu.sync_copy(dst_hbm.at[pl.ds(base, CE)], didx)
            _mask_dloc(didx, dloc, lo)
            if gather:
                pltpu.sync_copy(src_hbm.at[pl.ds(base, CE)], sidx)
                pltpu.sync_copy(g_hbm.at[sidx], gbuf)
            pltpu.sync_copy(gbuf, acc.at[dloc], add=True)

        plsc.subcore_barrier()
        _writeback(acc, out_hbm, bkt, s)
        plsc.subcore_barrier()


def _sc_gather_body(src_hbm, dst_hbm, g_hbm, out_hbm, *scratch):
    _sc_common(src_hbm, dst_hbm, g_hbm, out_hbm, *scratch, gather=True)


def _sc_deg_body(src_hbm, dst_hbm, out_hbm, *scratch):
    _sc_common(src_hbm, dst_hbm, None, out_hbm, *scratch, gather=False)


_SCRATCH = [
    pltpu.VMEM((CE,), jnp.int32),
    pltpu.VMEM((CE,), jnp.int32),
    pltpu.VMEM((CE,), jnp.int32),
    pltpu.VMEM((CE, DH), jnp.float32),
    pltpu.VMEM((256, DH), jnp.float32),
    pltpu.VMEM_SHARED((ACC_ROWS, DH), jnp.float32),
]


def _sc_msgpass(src, dst, g):
    fn = pl.kernel(_sc_gather_body,
                   out_type=jax.ShapeDtypeStruct((N, DH), jnp.float32),
                   mesh=_mesh, scratch_types=_SCRATCH)
    return fn(src, dst, g)


def _sc_deg(src, dst):
    fn = pl.kernel(_sc_deg_body,
                   out_type=jax.ShapeDtypeStruct((N, DH), jnp.float32),
                   mesh=_mesh, scratch_types=_SCRATCH)
    return fn(src, dst)


BLK = 2000
NBLK = N // BLK


def _tc_h0_body(x_ref, w_ref, o_ref):
    o_ref[...] = jnp.dot(x_ref[...], w_ref[...],
                         preferred_element_type=jnp.float32)


def _tc_h0(x, W0):
    return pl.pallas_call(
        _tc_h0_body,
        grid=(NBLK,),
        in_specs=[pl.BlockSpec((BLK, D_IN), lambda i: (i, 0)),
                  pl.BlockSpec((D_IN, DH), lambda i: (0, 0))],
        out_specs=pl.BlockSpec((BLK, DH), lambda i: (i, 0)),
        out_shape=jax.ShapeDtypeStruct((N, DH), jnp.float32),
    )(x, W0)


def _tc_dinv_g0_body(deg_ref, h_ref, dinv_ref, g_ref):
    dinv = lax.rsqrt(deg_ref[:, 0:1] + 1.0)
    dinv_ref[...] = dinv
    g_ref[...] = dinv * h_ref[...]


def _tc_dinv_g0(deg2d, h0):
    return pl.pallas_call(
        _tc_dinv_g0_body,
        grid=(NBLK,),
        in_specs=[pl.BlockSpec((BLK, DH), lambda i: (i, 0)),
                  pl.BlockSpec((BLK, DH), lambda i: (i, 0))],
        out_specs=[pl.BlockSpec((BLK, 1), lambda i: (i, 0)),
                   pl.BlockSpec((BLK, DH), lambda i: (i, 0))],
        out_shape=[jax.ShapeDtypeStruct((N, 1), jnp.float32),
                   jax.ShapeDtypeStruct((N, DH), jnp.float32)],
    )(deg2d, h0)


def _tc_layer_body(sp_ref, g_ref, dinv_ref, b_ref, w_ref, o_ref, *, last):
    dinv = dinv_ref[...]
    a = jnp.tanh(dinv * (sp_ref[...] + g_ref[...]) + b_ref[...])
    if last:
        o_ref[...] = a
    else:
        o_ref[...] = dinv * jnp.dot(a, w_ref[...],
                                    preferred_element_type=jnp.float32)


def _tc_layer(sp, g, dinv, b, Wnext, *, last):
    return pl.pallas_call(
        functools.partial(_tc_layer_body, last=last),
        grid=(NBLK,),
        in_specs=[pl.BlockSpec((BLK, DH), lambda i: (i, 0)),
                  pl.BlockSpec((BLK, DH), lambda i: (i, 0)),
                  pl.BlockSpec((BLK, 1), lambda i: (i, 0)),
                  pl.BlockSpec((1, DH), lambda i: (0, 0)),
                  pl.BlockSpec((DH, DH), lambda i: (0, 0))],
        out_specs=pl.BlockSpec((BLK, DH), lambda i: (i, 0)),
        out_shape=jax.ShapeDtypeStruct((N, DH), jnp.float32),
    )(sp, g, dinv, b, Wnext)


def _tc_pool_body(bat_smem, a_ref, bat_ref, hmax_ref, hsum_ref, cnt_ref):
    @pl.when(pl.program_id(0) == 0)
    def _():
        hmax_ref[...] = jnp.full((G, DH), -1e30, jnp.float32)
        hsum_ref[...] = jnp.zeros((G, DH), jnp.float32)
        cnt_ref[...] = jnp.zeros((G, 1), jnp.float32)

    a = a_ref[...]
    bcol = bat_ref[...]
    gmin = bat_smem[0]
    gmax = bat_smem[BLK - 1]

    def body(gg, _):
        m = bcol == gg
        bmax = jnp.max(jnp.where(m, a, -1e30), axis=0, keepdims=True)
        bsum = jnp.sum(jnp.where(m, a, 0.0), axis=0, keepdims=True)
        bcnt = jnp.sum(m.astype(jnp.float32), axis=0, keepdims=True)
        hmax_ref[pl.ds(gg, 1), :] = jnp.maximum(hmax_ref[pl.ds(gg, 1), :], bmax)
        hsum_ref[pl.ds(gg, 1), :] = hsum_ref[pl.ds(gg, 1), :] + bsum
        cnt_ref[pl.ds(gg, 1), :] = cnt_ref[pl.ds(gg, 1), :] + bcnt
        return 0

    lax.fori_loop(gmin, gmax + 1, body, 0)


def _tc_pool(a4, batch):
    return pl.pallas_call(
        _tc_pool_body,
        grid=(NBLK,),
        in_specs=[pl.BlockSpec((BLK,), lambda i: (i,),
                               memory_space=pltpu.SMEM),
                  pl.BlockSpec((BLK, DH), lambda i: (i, 0)),
                  pl.BlockSpec((BLK, 1), lambda i: (i, 0))],
        out_specs=[pl.BlockSpec((G, DH), lambda i: (0, 0)),
                   pl.BlockSpec((G, DH), lambda i: (0, 0)),
                   pl.BlockSpec((G, 1), lambda i: (0, 0))],
        out_shape=[jax.ShapeDtypeStruct((G, DH), jnp.float32),
                   jax.ShapeDtypeStruct((G, DH), jnp.float32),
                   jax.ShapeDtypeStruct((G, 1), jnp.float32)],
    )(batch, a4, batch.reshape(N, 1))


def _tc_final_body(hmax_ref, hsum_ref, cnt_ref, wmax_ref, wmean_ref, bout_ref,
                   o_ref):
    cnt = cnt_ref[...]
    hmax = jnp.where(cnt > 0, hmax_ref[...], 0.0)
    hmean = hsum_ref[...] / jnp.maximum(cnt, 1.0)
    o_ref[...] = (jnp.dot(hmax, wmax_ref[...],
                          preferred_element_type=jnp.float32)
                  + jnp.dot(hmean, wmean_ref[...],
                            preferred_element_type=jnp.float32)
                  + bout_ref[...])


def _tc_final(hmax, hsum, cnt, Wmax, Wmean, bout2d):
    return pl.pallas_call(
        _tc_final_body,
        out_shape=jax.ShapeDtypeStruct((G, 1), jnp.float32),
    )(hmax, hsum, cnt, Wmax, Wmean, bout2d)


def kernel(x, edge_index, batch, W0, b0, W1, b1, W2, b2, W3, b3, Wout, bout):
    src = edge_index[0]
    dst = edge_index[1]
    deg2d = _sc_deg(src, dst)
    h0 = _tc_h0(x, W0)
    dinv, g = _tc_dinv_g0(deg2d, h0)
    bs = [b0.reshape(1, DH), b1.reshape(1, DH), b2.reshape(1, DH),
          b3.reshape(1, DH)]
    Ws = [W1, W2, W3, W3]
    for l in range(4):
        sp = _sc_msgpass(src, dst, g)
        g = _tc_layer(sp, g, dinv, bs[l], Ws[l], last=(l == 3))
    hmax, hsum, cnt = _tc_pool(g, batch)
    return _tc_final(hmax, hsum, cnt, Wout[:DH], Wout[DH:],
                     bout.reshape(1, 1))


# speedup vs baseline: 4.2010x; 4.2010x over previous
import functools

import jax
import jax.numpy as jnp
from jax import lax
from jax.experimental import pallas as pl
from jax.experimental.pallas import tpu as pltpu
from jax.experimental.pallas import tpu_sc as plsc

N = 100000
E = 1600000
G = 512
D_IN = 9
DH = 64

NC = 2
NS = 16
LANES = 16

NBKT = 4
BK = N // NBKT
ACC_ROWS = 25088
SUB_ROWS = ACC_ROWS // NS
CE = 160
EPS = E // NS
NCH = EPS // CE

_mesh = plsc.VectorSubcoreMesh(core_axis_name="c", subcore_axis_name="s")


def _zero_acc(acc, zbuf, s):
    lo = s * SUB_ROWS

    @pl.loop(0, SUB_ROWS // 32)
    def _(k):
        pltpu.sync_copy(zbuf, acc.at[pl.ds(lo + k * 32, 32)])


def _fill(buf, value):
    v = jnp.full((LANES,), value, jnp.float32)

    @pl.loop(0, buf.shape[0])
    def _(r):
        for q in range(DH // LANES):
            buf[r, pl.ds(q * LANES, LANES)] = v


def _writeback(acc, out_hbm, bkt, s):
    lo = jnp.minimum(s * SUB_ROWS, BK - SUB_ROWS)
    pltpu.sync_copy(acc.at[pl.ds(lo, SUB_ROWS)],
                    out_hbm.at[pl.ds(bkt * BK + lo, SUB_ROWS)])


def _mask_dloc(didx, dloc, lo):
    @pl.loop(0, CE // LANES)
    def _(i):
        dv = didx[pl.ds(i * LANES, LANES)]
        inb = (dv >= lo) & (dv < lo + BK)
        dloc[pl.ds(i * LANES, LANES)] = jnp.where(inb, dv - lo, BK + (dv & 63))


def _sc_common(src_hbm, dst_hbm, g_hbm, out_hbm, sidx, didx, dloc, gbuf,
               zbuf, acc, *, gather):
    c = lax.axis_index("c")
    s = lax.axis_index("s")
    if not gather:
        _fill(gbuf, 1.0)
    _fill(zbuf, 0.0)
    for p in range(NC):
        bkt = NC * c + p
        lo = bkt * BK
        _zero_acc(acc, zbuf, s)
        plsc.subcore_barrier()

        @pl.loop(0, NCH)
        def _(ch):
            base = s * EPS + ch * CE
            pltpu.sync_copy(dst_hbm.at[pl.ds(base, CE)], didx)
            _mask_dloc(didx, dloc, lo)
            if gather:
                pltpu.sync_copy(src_hbm.at[pl.ds(base, CE)], sidx)
                pltpu.sync_copy(g_hbm.at[sidx], gbuf)
            pltpu.sync_copy(gbuf, acc.at[dloc], add=True)

        plsc.subcore_barrier()
        _writeback(acc, out_hbm, bkt, s)
        plsc.subcore_barrier()


def _sc_gather_body(src_hbm, dst_hbm, g_hbm, out_hbm, *scratch):
    _sc_common(src_hbm, dst_hbm, g_hbm, out_hbm, *scratch, gather=True)


def _sc_deg_body(src_hbm, dst_hbm, out_hbm, *scratch):
    _sc_common(src_hbm, dst_hbm, None, out_hbm, *scratch, gather=False)


_SC_PARAMS = pltpu.CompilerParams(use_tc_tiling_on_sc=False)

_SCRATCH = [
    pltpu.VMEM((CE,), jnp.int32),
    pltpu.VMEM((CE,), jnp.int32),
    pltpu.VMEM((CE,), jnp.int32),
    pltpu.VMEM((CE, DH), jnp.float32),
    pltpu.VMEM((32, DH), jnp.float32),
    pltpu.VMEM_SHARED((ACC_ROWS, DH), jnp.float32),
]


def _sc_msgpass(src, dst, g):
    fn = pl.kernel(_sc_gather_body,
                   out_type=jax.ShapeDtypeStruct((N, DH), jnp.float32),
                   mesh=_mesh, scratch_types=_SCRATCH,
                   compiler_params=_SC_PARAMS)
    return fn(src, dst, g)


def _sc_deg(src, dst):
    fn = pl.kernel(_sc_deg_body,
                   out_type=jax.ShapeDtypeStruct((N, DH), jnp.float32),
                   mesh=_mesh, scratch_types=_SCRATCH,
                   compiler_params=_SC_PARAMS)
    return fn(src, dst)


BLK = 2000
NBLK = N // BLK


def _tc_h0_body(x_ref, w_ref, o_ref):
    o_ref[...] = jnp.dot(x_ref[...], w_ref[...],
                         preferred_element_type=jnp.float32)


def _tc_h0(x, W0):
    return pl.pallas_call(
        _tc_h0_body,
        grid=(NBLK,),
        in_specs=[pl.BlockSpec((BLK, D_IN), lambda i: (i, 0)),
                  pl.BlockSpec((D_IN, DH), lambda i: (0, 0))],
        out_specs=pl.BlockSpec((BLK, DH), lambda i: (i, 0)),
        out_shape=jax.ShapeDtypeStruct((N, DH), jnp.float32),
    )(x, W0)


def _tc_dinv_g0_body(deg_ref, h_ref, dinv_ref, g_ref):
    dinv = lax.rsqrt(deg_ref[:, 0:1] + 1.0)
    dinv_ref[...] = dinv
    g_ref[...] = dinv * h_ref[...]


def _tc_dinv_g0(deg2d, h0):
    return pl.pallas_call(
        _tc_dinv_g0_body,
        grid=(NBLK,),
        in_specs=[pl.BlockSpec((BLK, DH), lambda i: (i, 0)),
                  pl.BlockSpec((BLK, DH), lambda i: (i, 0))],
        out_specs=[pl.BlockSpec((BLK, 1), lambda i: (i, 0)),
                   pl.BlockSpec((BLK, DH), lambda i: (i, 0))],
        out_shape=[jax.ShapeDtypeStruct((N, 1), jnp.float32),
                   jax.ShapeDtypeStruct((N, DH), jnp.float32)],
    )(deg2d, h0)


def _tc_layer_body(sp_ref, g_ref, dinv_ref, b_ref, w_ref, o_ref, *, last):
    dinv = dinv_ref[...]
    a = jnp.tanh(dinv * (sp_ref[...] + g_ref[...]) + b_ref[...])
    if last:
        o_ref[...] = a
    else:
        o_ref[...] = dinv * jnp.dot(a, w_ref[...],
                                    preferred_element_type=jnp.float32)


def _tc_layer(sp, g, dinv, b, Wnext, *, last):
    return pl.pallas_call(
        functools.partial(_tc_layer_body, last=last),
        grid=(NBLK,),
        in_specs=[pl.BlockSpec((BLK, DH), lambda i: (i, 0)),
                  pl.BlockSpec((BLK, DH), lambda i: (i, 0)),
                  pl.BlockSpec((BLK, 1), lambda i: (i, 0)),
                  pl.BlockSpec((1, DH), lambda i: (0, 0)),
                  pl.BlockSpec((DH, DH), lambda i: (0, 0))],
        out_specs=pl.BlockSpec((BLK, DH), lambda i: (i, 0)),
        out_shape=jax.ShapeDtypeStruct((N, DH), jnp.float32),
    )(sp, g, dinv, b, Wnext)


def _tc_pool_body(bat_smem, a_ref, bat_ref, hmax_ref, hsum_ref, cnt_ref):
    @pl.when(pl.program_id(0) == 0)
    def _():
        hmax_ref[...] = jnp.full((G, DH), -1e30, jnp.float32)
        hsum_ref[...] = jnp.zeros((G, DH), jnp.float32)
        cnt_ref[...] = jnp.zeros((G, 1), jnp.float32)

    a = a_ref[...]
    bcol = bat_ref[...]
    gmin = bat_smem[0, 0, 0]
    gmax = bat_smem[0, 0, BLK - 1]

    def body(gg, _):
        m = bcol == gg
        bmax = jnp.max(jnp.where(m, a, -1e30), axis=0, keepdims=True)
        bsum = jnp.sum(jnp.where(m, a, 0.0), axis=0, keepdims=True)
        bcnt = jnp.sum(m.astype(jnp.float32), axis=0, keepdims=True)
        hmax_ref[pl.ds(gg, 1), :] = jnp.maximum(hmax_ref[pl.ds(gg, 1), :], bmax)
        hsum_ref[pl.ds(gg, 1), :] = hsum_ref[pl.ds(gg, 1), :] + bsum
        cnt_ref[pl.ds(gg, 1), :] = cnt_ref[pl.ds(gg, 1), :] + bcnt
        return 0

    lax.fori_loop(gmin, gmax + 1, body, 0)


def _tc_pool(a4, batch):
    return pl.pallas_call(
        _tc_pool_body,
        grid=(NBLK,),
        in_specs=[pl.BlockSpec((1, 1, BLK), lambda i: (i, 0, 0),
                               memory_space=pltpu.SMEM),
                  pl.BlockSpec((BLK, DH), lambda i: (i, 0)),
                  pl.BlockSpec((BLK, 1), lambda i: (i, 0))],
        out_specs=[pl.BlockSpec((G, DH), lambda i: (0, 0)),
                   pl.BlockSpec((G, DH), lambda i: (0, 0)),
                   pl.BlockSpec((G, 1), lambda i: (0, 0))],
        out_shape=[jax.ShapeDtypeStruct((G, DH), jnp.float32),
                   jax.ShapeDtypeStruct((G, DH), jnp.float32),
                   jax.ShapeDtypeStruct((G, 1), jnp.float32)],
    )(batch.reshape(NBLK, 1, BLK), a4, batch.reshape(N, 1))


def _tc_final_body(hmax_ref, hsum_ref, cnt_ref, wmax_ref, wmean_ref, bout_ref,
                   o_ref):
    cnt = cnt_ref[...]
    hmax = jnp.where(cnt > 0, hmax_ref[...], 0.0)
    hmean = hsum_ref[...] / jnp.maximum(cnt, 1.0)
    o_ref[...] = (jnp.dot(hmax, wmax_ref[...],
                          preferred_element_type=jnp.float32)
                  + jnp.dot(hmean, wmean_ref[...],
                            preferred_element_type=jnp.float32)
                  + bout_ref[...])


def _tc_final(hmax, hsum, cnt, Wmax, Wmean, bout2d):
    return pl.pallas_call(
        _tc_final_body,
        out_shape=jax.ShapeDtypeStruct((G, 1), jnp.float32),
    )(hmax, hsum, cnt, Wmax, Wmean, bout2d)


def kernel(x, edge_index, batch, W0, b0, W1, b1, W2, b2, W3, b3, Wout, bout):
    src = edge_index[0]
    dst = edge_index[1]
    deg2d = _sc_deg(src, dst)
    h0 = _tc_h0(x, W0)
    dinv, g = _tc_dinv_g0(deg2d, h0)
    bs = [b0.reshape(1, DH), b1.reshape(1, DH), b2.reshape(1, DH),
          b3.reshape(1, DH)]
    Ws = [W1, W2, W3, W3]
    for l in range(4):
        sp = _sc_msgpass(src, dst, g)
        g = _tc_layer(sp, g, dinv, bs[l], Ws[l], last=(l == 3))
    hmax, hsum, cnt = _tc_pool(g, batch)
    return _tc_final(hmax, hsum, cnt, Wout[:DH], Wout[DH:],
                     bout.reshape(1, 1))


# speedup vs baseline: 18.0332x; 4.2926x over previous
import functools

import jax
import jax.numpy as jnp
from jax import lax
from jax.experimental import pallas as pl
from jax.experimental.pallas import tpu as pltpu
from jax.experimental.pallas import tpu_sc as plsc

N = 100000
E = 1600000
G = 512
D_IN = 9
DH = 64

NC = 2
NS = 16
LANES = 16

NBKT = 4
BK = N // NBKT
ACC_ROWS = 25088
SUB_ROWS = ACC_ROWS // NS
CE = 400
EPS = E // NS
NCH = EPS // CE
NW = NC * NS
EPW = E // NW
PCH = EPW // CE
CAPW = 50400
STG = 816
DEGP = 100096
DSUB = DEGP // NS

_mesh = plsc.VectorSubcoreMesh(core_axis_name="c", subcore_axis_name="s")


def _zero_acc(acc, zbuf, s):
    lo = s * SUB_ROWS

    @pl.loop(0, SUB_ROWS // 32)
    def _(k):
        pltpu.sync_copy(zbuf, acc.at[pl.ds(lo + k * 32, 32)])


def _fill(buf, value):
    v = jnp.full((LANES,), value, jnp.float32)

    @pl.loop(0, buf.shape[0])
    def _(r):
        for q in range(DH // LANES):
            buf[r, pl.ds(q * LANES, LANES)] = v


def _writeback(acc, out_hbm, bkt, s):
    lo = jnp.minimum(s * SUB_ROWS, BK - SUB_ROWS)
    pltpu.sync_copy(acc.at[pl.ds(lo, SUB_ROWS)],
                    out_hbm.at[pl.ds(bkt * BK + lo, SUB_ROWS)])


def _mask_dloc(didx, dloc, lo):
    @pl.loop(0, CE // LANES)
    def _(i):
        dv = didx[pl.ds(i * LANES, LANES)]
        inb = (dv >= lo) & (dv < lo + BK)
        dloc[pl.ds(i * LANES, LANES)] = jnp.where(inb, dv - lo,
                                                  BK + (dv & 63))


def _sc_common(src_hbm, dst_hbm, g_hbm, out_hbm,
               sidx, didx, dloc, gbuf, zbuf, acc, *, gather):
    c = lax.axis_index("c")
    s = lax.axis_index("s")
    if not gather:
        _fill(gbuf, 1.0)
    _fill(zbuf, 0.0)
    for p in range(NC):
        bkt = NC * c + p
        lo = bkt * BK
        _zero_acc(acc, zbuf, s)
        plsc.subcore_barrier()

        @pl.loop(0, NCH)
        def _(ch):
            base = s * EPS + ch * CE
            pltpu.sync_copy(dst_hbm.at[pl.ds(base, CE)], didx)
            _mask_dloc(didx, dloc, lo)
            if gather:
                pltpu.sync_copy(src_hbm.at[pl.ds(base, CE)], sidx)
                pltpu.sync_copy(g_hbm.at[sidx], gbuf)
            pltpu.sync_copy(gbuf, acc.at[dloc], add=True)

        plsc.subcore_barrier()
        _writeback(acc, out_hbm, bkt, s)
        plsc.subcore_barrier()


def _sc_gather_body(src_hbm, dst_hbm, g_hbm, out_hbm, *scratch):
    _sc_common(src_hbm, dst_hbm, g_hbm, out_hbm, *scratch, gather=True)


def _sc_deg_body(src_hbm, dst_hbm, out_hbm, *scratch):
    _sc_common(src_hbm, dst_hbm, None, out_hbm, *scratch, gather=False)


_SC_PARAMS = pltpu.CompilerParams(use_tc_tiling_on_sc=False,
                                  needs_layout_passes=False)

_SCRATCH = [
    pltpu.VMEM((CE,), jnp.int32),
    pltpu.VMEM((CE,), jnp.int32),
    pltpu.VMEM((CE,), jnp.int32),
    pltpu.VMEM((CE, DH), jnp.float32),
    pltpu.VMEM((32, DH), jnp.float32),
    pltpu.VMEM_SHARED((ACC_ROWS, DH), jnp.float32),
]


def _sc_msgpass(src, dst, g):
    fn = pl.kernel(_sc_gather_body,
                   out_type=jax.ShapeDtypeStruct((N, DH), jnp.float32),
                   mesh=_mesh, scratch_types=_SCRATCH,
                   compiler_params=_SC_PARAMS)
    return fn(src, dst, g)


def _sc_deg(src, dst):
    fn = pl.kernel(_sc_deg_body,
                   out_type=jax.ShapeDtypeStruct((N, DH), jnp.float32),
                   mesh=_mesh, scratch_types=_SCRATCH,
                   compiler_params=_SC_PARAMS)
    return fn(src, dst)


def _fill1d(buf, value, n):
    v = jnp.full((LANES,), value, jnp.float32)

    @pl.loop(0, n // LANES)
    def _(r):
        buf[pl.ds(r * LANES, LANES)] = v


def _iota16():
    return lax.iota(jnp.int32, LANES)


def _extract(vec, e):
    return jnp.sum(jnp.where(_iota16() == e, vec, 0))


def _sc_part_body(src_hbm, dst_hbm, bsrc_hbm, bdl_hbm, nchk_hbm, deg_hbm,
                  sbuf, dbuf, ones, zb1, s0, s1, s2, s3, d0, d1, d2, d3,
                  nvec_buf, deg_sp):
    c = lax.axis_index("c")
    s = lax.axis_index("s")
    w = 2 * s + c
    stag_s = [s0, s1, s2, s3]
    stag_d = [d0, d1, d2, d3]
    _fill1d(ones, 1.0, CE)
    _fill1d(zb1, 0.0, 2048)
    for k in range(3):
        pltpu.sync_copy(zb1, deg_sp.at[pl.ds(s * DSUB + k * 2048, 2048)])
    pltpu.sync_copy(zb1.at[pl.ds(0, DSUB - 3 * 2048)],
                    deg_sp.at[pl.ds(s * DSUB + 3 * 2048, DSUB - 3 * 2048)])
    plsc.subcore_barrier()

    def chunk_body(ch, carry):
        offs = list(carry[0:4])
        hs = list(carry[4:8])
        base = w * EPW + ch * CE
        pltpu.sync_copy(src_hbm.at[pl.ds(base, CE)], sbuf)
        pltpu.sync_copy(dst_hbm.at[pl.ds(base, CE)], dbuf)
        pltpu.sync_copy(ones, deg_sp.at[dbuf], add=True)

        def vec_body(i, offs4):
            sv = sbuf[pl.ds(i * LANES, LANES)]
            dv = dbuf[pl.ds(i * LANES, LANES)]
            new = []
            for b in range(NBKT):
                lob = b * BK
                m = (dv >= lob) & (dv < lob + BK)
                mint = jnp.where(m, 1, 0)
                idxv = offs4[b] + plsc.cumsum(mint) - 1
                plsc.store_scatter(stag_s[b], [idxv], sv, mask=m)
                plsc.store_scatter(stag_d[b], [idxv], dv - lob, mask=m)
                new.append(offs4[b] + jnp.sum(mint))
            return tuple(new)

        offs = list(lax.fori_loop(0, CE // LANES, vec_body, tuple(offs)))
        for b in range(NBKT):
            full = offs[b] >= CE
            regbase = (w * NBKT + b) * CAPW

            @pl.when(full)
            def _():
                pltpu.sync_copy(stag_s[b].at[pl.ds(0, CE)],
                                bsrc_hbm.at[pl.ds(regbase + hs[b] * CE, CE)])
                pltpu.sync_copy(stag_d[b].at[pl.ds(0, CE)],
                                bdl_hbm.at[pl.ds(regbase + hs[b] * CE, CE)])
                for t in range(CE // LANES + 1):
                    stag_s[b][pl.ds(t * LANES, LANES)] = \
                        stag_s[b][pl.ds(CE + t * LANES, LANES)]
                    stag_d[b][pl.ds(t * LANES, LANES)] = \
                        stag_d[b][pl.ds(CE + t * LANES, LANES)]
            offs[b] = offs[b] - jnp.where(full, CE, 0)
            hs[b] = hs[b] + jnp.where(full, 1, 0)
        return tuple(offs) + tuple(hs)

    carry = lax.fori_loop(0, PCH, chunk_body,
                          (jnp.int32(0),) * 4 + (jnp.int32(0),) * 4)
    offs, hs = carry[0:4], carry[4:8]
    nvec = jnp.zeros((LANES,), jnp.int32)
    for b in range(NBKT):
        regbase = (w * NBKT + b) * CAPW
        for t in range(CE // LANES):
            dummy_s = (_iota16() + t * LANES) & 1023
            dummy_d = BK + ((_iota16() + t * LANES) & 63)
            stag_s[b][pl.ds(offs[b] + t * LANES, LANES)] = dummy_s
            stag_d[b][pl.ds(offs[b] + t * LANES, LANES)] = dummy_d
        pltpu.sync_copy(stag_s[b].at[pl.ds(0, CE)],
                        bsrc_hbm.at[pl.ds(regbase + hs[b] * CE, CE)])
        pltpu.sync_copy(stag_d[b].at[pl.ds(0, CE)],
                        bdl_hbm.at[pl.ds(regbase + hs[b] * CE, CE)])
        nvec = jnp.where(_iota16() == b, hs[b] + 1, nvec)
    nvec_buf[pl.ds(0, LANES)] = nvec
    pltpu.sync_copy(nvec_buf.at[pl.ds(0, 8)],
                    nchk_hbm.at[pl.ds(s * 16 + c * 8, 8)])
    plsc.subcore_barrier()
    pltpu.sync_copy(deg_sp.at[pl.ds(s * DSUB, DSUB)],
                    deg_hbm.at[c, pl.ds(s * DSUB, DSUB)])


def _sc_part(src, dst):
    scratch = [
        pltpu.VMEM((CE,), jnp.int32),
        pltpu.VMEM((CE,), jnp.int32),
        pltpu.VMEM((CE,), jnp.float32),
        pltpu.VMEM((2048,), jnp.float32),
    ] + [pltpu.VMEM((STG,), jnp.int32) for _ in range(8)] + [
        pltpu.VMEM((LANES,), jnp.int32),
        pltpu.VMEM_SHARED((DEGP,), jnp.float32),
    ]
    fn = pl.kernel(
        _sc_part_body,
        out_type=(jax.ShapeDtypeStruct((NW * NBKT * CAPW,), jnp.int32),
                  jax.ShapeDtypeStruct((NW * NBKT * CAPW,), jnp.int32),
                  jax.ShapeDtypeStruct((NS * 16,), jnp.int32),
                  jax.ShapeDtypeStruct((NC, DEGP), jnp.float32)),
        mesh=_mesh, scratch_types=scratch, compiler_params=_SC_PARAMS)
    return fn(src, dst)


def _sc_binned_body(bsrc_hbm, bdl_hbm, nchk_hbm, g_hbm, out_hbm,
                    sidx, didx, dloc, gbuf, zbuf, nvec_buf, acc):
    del dloc
    c = lax.axis_index("c")
    s = lax.axis_index("s")
    _fill(zbuf, 0.0)
    pltpu.sync_copy(nchk_hbm.at[pl.ds(s * 16, 16)], nvec_buf)
    nvec = nvec_buf[pl.ds(0, LANES)]
    for p in range(NC):
        bkt = NC * c + p
        nch0 = _extract(nvec, bkt)
        nch1 = _extract(nvec, 8 + bkt)
        base0 = ((2 * s + 0) * NBKT + bkt) * CAPW
        base1 = ((2 * s + 1) * NBKT + bkt) * CAPW
        _zero_acc(acc, zbuf, s)
        plsc.subcore_barrier()

        @pl.loop(0, nch0 + nch1)
        def _(t):
            addr = jnp.where(t < nch0, base0 + t * CE,
                             base1 + (t - nch0) * CE)
            pltpu.sync_copy(bsrc_hbm.at[pl.ds(addr, CE)], sidx)
            pltpu.sync_copy(bdl_hbm.at[pl.ds(addr, CE)], didx)
            pltpu.sync_copy(g_hbm.at[sidx], gbuf)
            pltpu.sync_copy(gbuf, acc.at[didx], add=True)

        plsc.subcore_barrier()
        _writeback(acc, out_hbm, bkt, s)
        plsc.subcore_barrier()


def _sc_binned(bsrc, bdl, nchk, g):
    fn = pl.kernel(_sc_binned_body,
                   out_type=jax.ShapeDtypeStruct((N, DH), jnp.float32),
                   mesh=_mesh, scratch_types=_SCRATCH[:5] + [
                       pltpu.VMEM((LANES,), jnp.int32)] + _SCRATCH[5:],
                   compiler_params=_SC_PARAMS)
    return fn(bsrc, bdl, nchk, g)


BLK = 2000
NBLK = N // BLK


def _tc_h0_body(x_ref, w_ref, o_ref):
    o_ref[...] = jnp.dot(x_ref[...], w_ref[...],
                         preferred_element_type=jnp.float32)


def _tc_h0(x, W0):
    return pl.pallas_call(
        _tc_h0_body,
        grid=(NBLK,),
        in_specs=[pl.BlockSpec((BLK, D_IN), lambda i: (i, 0)),
                  pl.BlockSpec((D_IN, DH), lambda i: (0, 0))],
        out_specs=pl.BlockSpec((BLK, DH), lambda i: (i, 0)),
        out_shape=jax.ShapeDtypeStruct((N, DH), jnp.float32),
    )(x, W0)


def _tc_dinv_g0_body(d0_ref, d1_ref, h_ref, dinv_ref, g_ref):
    dinv = lax.rsqrt(d0_ref[...] + d1_ref[...] + 1.0)
    dinv_ref[...] = dinv
    g_ref[...] = dinv * h_ref[...]


def _tc_dinv_g0(d0, d1, h0):
    return pl.pallas_call(
        _tc_dinv_g0_body,
        grid=(NBLK,),
        in_specs=[pl.BlockSpec((BLK, 1), lambda i: (i, 0)),
                  pl.BlockSpec((BLK, 1), lambda i: (i, 0)),
                  pl.BlockSpec((BLK, DH), lambda i: (i, 0))],
        out_specs=[pl.BlockSpec((BLK, 1), lambda i: (i, 0)),
                   pl.BlockSpec((BLK, DH), lambda i: (i, 0))],
        out_shape=[jax.ShapeDtypeStruct((N, 1), jnp.float32),
                   jax.ShapeDtypeStruct((N, DH), jnp.float32)],
    )(d0, d1, h0)


def _tc_layer_body(sp_ref, g_ref, dinv_ref, b_ref, w_ref, o_ref, *, last):
    dinv = dinv_ref[...]
    a = jnp.tanh(dinv * (sp_ref[...] + g_ref[...]) + b_ref[...])
    if last:
        o_ref[...] = a
    else:
        o_ref[...] = dinv * jnp.dot(a, w_ref[...],
                                    preferred_element_type=jnp.float32)


def _tc_layer(sp, g, dinv, b, Wnext, *, last):
    return pl.pallas_call(
        functools.partial(_tc_layer_body, last=last),
        grid=(NBLK,),
        in_specs=[pl.BlockSpec((BLK, DH), lambda i: (i, 0)),
                  pl.BlockSpec((BLK, DH), lambda i: (i, 0)),
                  pl.BlockSpec((BLK, 1), lambda i: (i, 0)),
                  pl.BlockSpec((1, DH), lambda i: (0, 0)),
                  pl.BlockSpec((DH, DH), lambda i: (0, 0))],
        out_specs=pl.BlockSpec((BLK, DH), lambda i: (i, 0)),
        out_shape=jax.ShapeDtypeStruct((N, DH), jnp.float32),
    )(sp, g, dinv, b, Wnext)


def _tc_pool_body(bat_smem, a_ref, bat_ref, hmax_ref, hsum_ref, cnt_ref):
    @pl.when(pl.program_id(0) == 0)
    def _():
        hmax_ref[...] = jnp.full((G, DH), -1e30, jnp.float32)
        hsum_ref[...] = jnp.zeros((G, DH), jnp.float32)
        cnt_ref[...] = jnp.zeros((G, 1), jnp.float32)

    a = a_ref[...]
    bcol = bat_ref[...]
    gmin = bat_smem[0, 0, 0]
    gmax = bat_smem[0, 0, BLK - 1]

    def body(gg, _):
        m = bcol == gg
        bmax = jnp.max(jnp.where(m, a, -1e30), axis=0, keepdims=True)
        bsum = jnp.sum(jnp.where(m, a, 0.0), axis=0, keepdims=True)
        bcnt = jnp.sum(m.astype(jnp.float32), axis=0, keepdims=True)
        hmax_ref[pl.ds(gg, 1), :] = jnp.maximum(hmax_ref[pl.ds(gg, 1), :], bmax)
        hsum_ref[pl.ds(gg, 1), :] = hsum_ref[pl.ds(gg, 1), :] + bsum
        cnt_ref[pl.ds(gg, 1), :] = cnt_ref[pl.ds(gg, 1), :] + bcnt
        return 0

    lax.fori_loop(gmin, gmax + 1, body, 0)


def _tc_pool(a4, batch):
    return pl.pallas_call(
        _tc_pool_body,
        grid=(NBLK,),
        in_specs=[pl.BlockSpec((1, 1, BLK), lambda i: (i, 0, 0),
                               memory_space=pltpu.SMEM),
                  pl.BlockSpec((BLK, DH), lambda i: (i, 0)),
                  pl.BlockSpec((BLK, 1), lambda i: (i, 0))],
        out_specs=[pl.BlockSpec((G, DH), lambda i: (0, 0)),
                   pl.BlockSpec((G, DH), lambda i: (0, 0)),
                   pl.BlockSpec((G, 1), lambda i: (0, 0))],
        out_shape=[jax.ShapeDtypeStruct((G, DH), jnp.float32),
                   jax.ShapeDtypeStruct((G, DH), jnp.float32),
                   jax.ShapeDtypeStruct((G, 1), jnp.float32)],
    )(batch.reshape(NBLK, 1, BLK), a4, batch.reshape(N, 1))


def _tc_final_body(hmax_ref, hsum_ref, cnt_ref, wmax_ref, wmean_ref, bout_ref,
                   o_ref):
    cnt = cnt_ref[...]
    hmax = jnp.where(cnt > 0, hmax_ref[...], 0.0)
    hmean = hsum_ref[...] / jnp.maximum(cnt, 1.0)
    o_ref[...] = (jnp.dot(hmax, wmax_ref[...],
                          preferred_element_type=jnp.float32)
                  + jnp.dot(hmean, wmean_ref[...],
                            preferred_element_type=jnp.float32)
                  + bout_ref[...])


def _tc_final(hmax, hsum, cnt, Wmax, Wmean, bout2d):
    return pl.pallas_call(
        _tc_final_body,
        out_shape=jax.ShapeDtypeStruct((G, 1), jnp.float32),
    )(hmax, hsum, cnt, Wmax, Wmean, bout2d)


def kernel(x, edge_index, batch, W0, b0, W1, b1, W2, b2, W3, b3, Wout, bout):
    src = edge_index[0]
    dst = edge_index[1]
    bsrc, bdl, nchk, degp = _sc_part(src, dst)
    h0 = _tc_h0(x, W0)
    dinv, g = _tc_dinv_g0(degp[0, :N].reshape(N, 1),
                          degp[1, :N].reshape(N, 1), h0)
    bs = [b0.reshape(1, DH), b1.reshape(1, DH), b2.reshape(1, DH),
          b3.reshape(1, DH)]
    Ws = [W1, W2, W3, W3]
    for l in range(4):
        sp = _sc_binned(bsrc, bdl, nchk, g)
        g = _tc_layer(sp, g, dinv, bs[l], Ws[l], last=(l == 3))
    hmax, hsum, cnt = _tc_pool(g, batch)
    return _tc_final(hmax, hsum, cnt, Wout[:DH], Wout[DH:],
                     bout.reshape(1, 1))


# speedup vs baseline: 20.1906x; 1.1196x over previous
import functools

import jax
import jax.numpy as jnp
from jax import lax
from jax.experimental import pallas as pl
from jax.experimental.pallas import tpu as pltpu
from jax.experimental.pallas import tpu_sc as plsc

N = 100000
E = 1600000
G = 512
D_IN = 9
DH = 64

NC = 2
NS = 16
LANES = 16

NBKT = 8
BK = N // NBKT
ACC_ROWS = 12608
SUB_ROWS = ACC_ROWS // NS
CE = 400
CEC = 800
NW = NC * NS
EPW = E // NW
PCH = EPW // CE
CAPW = 63 * CEC
STG = CEC + CE + 16
DEGP = 100096
DSUB = DEGP // NS

_mesh = plsc.VectorSubcoreMesh(core_axis_name="c", subcore_axis_name="s")


def _zero_acc(acc, zbuf, s):
    lo = s * SUB_ROWS

    @pl.loop(0, SUB_ROWS // 128)
    def _(k):
        pltpu.sync_copy(zbuf, acc.at[pl.ds(lo + k * 128, 128)])
    rem = SUB_ROWS % 128
    if rem:
        pltpu.sync_copy(zbuf.at[pl.ds(0, rem)],
                        acc.at[pl.ds(lo + SUB_ROWS - rem, rem)])


def _fill(buf, value):
    v = jnp.full((LANES,), value, jnp.float32)

    @pl.loop(0, buf.shape[0])
    def _(r):
        for q in range(DH // LANES):
            buf[r, pl.ds(q * LANES, LANES)] = v


def _writeback(acc, out_hbm, bkt, s):
    lo = jnp.minimum(s * SUB_ROWS, BK - SUB_ROWS)
    pltpu.sync_copy(acc.at[pl.ds(lo, SUB_ROWS)],
                    out_hbm.at[pl.ds(bkt * BK + lo, SUB_ROWS)])


_SC_PARAMS = pltpu.CompilerParams(use_tc_tiling_on_sc=False,
                                  needs_layout_passes=False)

_SCRATCH = [
    pltpu.VMEM((CEC,), jnp.int32),
    pltpu.VMEM((CEC,), jnp.int32),
    pltpu.VMEM((CEC,), jnp.int32),
    pltpu.VMEM((CEC, DH), jnp.float32),
    pltpu.VMEM((128, DH), jnp.float32),
    pltpu.VMEM_SHARED((ACC_ROWS, DH), jnp.float32),
]


def _fill1d(buf, value, n):
    v = jnp.full((LANES,), value, jnp.float32)

    @pl.loop(0, n // LANES)
    def _(r):
        buf[pl.ds(r * LANES, LANES)] = v


def _iota16():
    return lax.iota(jnp.int32, LANES)


def _extract(vec, e):
    return jnp.sum(jnp.where(_iota16() == e, vec, 0))


def _sc_part_body(ei_hbm, bsrc_hbm, bdl_hbm, nchk_hbm, deg_hbm,
                  sbuf, dbuf, ones, zb1, *rest):
    stag_s = list(rest[0:NBKT])
    stag_d = list(rest[NBKT:2 * NBKT])
    nvec_buf = rest[2 * NBKT]
    deg_sp = rest[2 * NBKT + 1]
    c = lax.axis_index("c")
    s = lax.axis_index("s")
    w = 2 * s + c
    _fill1d(ones, 1.0, CE)
    _fill1d(zb1, 0.0, 2048)
    for k in range(3):
        pltpu.sync_copy(zb1, deg_sp.at[pl.ds(s * DSUB + k * 2048, 2048)])
    pltpu.sync_copy(zb1.at[pl.ds(0, DSUB - 3 * 2048)],
                    deg_sp.at[pl.ds(s * DSUB + 3 * 2048, DSUB - 3 * 2048)])
    plsc.subcore_barrier()

    def chunk_body(ch, carry):
        offs = list(carry[0:NBKT])
        hs = list(carry[NBKT:2 * NBKT])
        base = w * EPW + ch * CE
        pltpu.sync_copy(ei_hbm.at[0, pl.ds(base, CE)], sbuf)
        pltpu.sync_copy(ei_hbm.at[1, pl.ds(base, CE)], dbuf)
        pltpu.sync_copy(ones, deg_sp.at[dbuf], add=True)

        def vec_body(i, offs4):
            sv = sbuf[pl.ds(i * LANES, LANES)]
            dv = dbuf[pl.ds(i * LANES, LANES)]
            new = []
            for b in range(NBKT):
                lob = b * BK
                m = (dv >= lob) & (dv < lob + BK)
                mint = jnp.where(m, 1, 0)
                idxv = offs4[b] + plsc.cumsum(mint) - 1
                plsc.store_scatter(stag_s[b], [idxv], sv, mask=m)
                plsc.store_scatter(stag_d[b], [idxv], dv - lob, mask=m)
                new.append(offs4[b] + jnp.sum(mint))
            return tuple(new)

        offs = list(lax.fori_loop(0, CE // LANES, vec_body, tuple(offs)))
        for b in range(NBKT):
            full = offs[b] >= CEC
            regbase = (w * NBKT + b) * CAPW

            @pl.when(full)
            def _():
                pltpu.sync_copy(stag_s[b].at[pl.ds(0, CEC)],
                                bsrc_hbm.at[pl.ds(regbase + hs[b] * CEC, CEC)])
                pltpu.sync_copy(stag_d[b].at[pl.ds(0, CEC)],
                                bdl_hbm.at[pl.ds(regbase + hs[b] * CEC, CEC)])
                for t in range(CE // LANES + 1):
                    stag_s[b][pl.ds(t * LANES, LANES)] = \
                        stag_s[b][pl.ds(CEC + t * LANES, LANES)]
                    stag_d[b][pl.ds(t * LANES, LANES)] = \
                        stag_d[b][pl.ds(CEC + t * LANES, LANES)]
            offs[b] = offs[b] - jnp.where(full, CEC, 0)
            hs[b] = hs[b] + jnp.where(full, 1, 0)
        return tuple(offs) + tuple(hs)

    carry = lax.fori_loop(0, PCH, chunk_body,
                          (jnp.int32(0),) * (2 * NBKT))
    offs, hs = carry[0:NBKT], carry[NBKT:2 * NBKT]
    nvec = jnp.zeros((LANES,), jnp.int32)
    for b in range(NBKT):
        regbase = (w * NBKT + b) * CAPW
        for t in range(CEC // LANES):
            dummy_s = (_iota16() + t * LANES) & 1023
            dummy_d = BK + ((_iota16() + t * LANES) & 63)
            pos = jnp.minimum(offs[b] + t * LANES, STG - LANES)
            stag_s[b][pl.ds(pos, LANES)] = dummy_s
            stag_d[b][pl.ds(pos, LANES)] = dummy_d
        pltpu.sync_copy(stag_s[b].at[pl.ds(0, CEC)],
                        bsrc_hbm.at[pl.ds(regbase + hs[b] * CEC, CEC)])
        pltpu.sync_copy(stag_d[b].at[pl.ds(0, CEC)],
                        bdl_hbm.at[pl.ds(regbase + hs[b] * CEC, CEC)])
        nvec = jnp.where(_iota16() == b, hs[b] + 1, nvec)
    nvec_buf[pl.ds(0, LANES)] = nvec
    pltpu.sync_copy(nvec_buf.at[pl.ds(0, 8)],
                    nchk_hbm.at[pl.ds(s * 16 + c * 8, 8)])
    plsc.subcore_barrier()
    pltpu.sync_copy(deg_sp.at[pl.ds(s * DSUB, DSUB)],
                    deg_hbm.at[c, pl.ds(s * DSUB, DSUB)])


def _sc_part(edge_index):
    scratch = [
        pltpu.VMEM((CE,), jnp.int32),
        pltpu.VMEM((CE,), jnp.int32),
        pltpu.VMEM((CE,), jnp.float32),
        pltpu.VMEM((2048,), jnp.float32),
    ] + [pltpu.VMEM((STG,), jnp.int32) for _ in range(2 * NBKT)] + [
        pltpu.VMEM((LANES,), jnp.int32),
        pltpu.VMEM_SHARED((DEGP,), jnp.float32),
    ]
    fn = pl.kernel(
        _sc_part_body,
        out_type=(jax.ShapeDtypeStruct((NW * NBKT * CAPW,), jnp.int32),
                  jax.ShapeDtypeStruct((NW * NBKT * CAPW,), jnp.int32),
                  jax.ShapeDtypeStruct((NS * 16,), jnp.int32),
                  jax.ShapeDtypeStruct((NC, DEGP), jnp.float32)),
        mesh=_mesh, scratch_types=scratch, compiler_params=_SC_PARAMS)
    return fn(edge_index)


def _sc_binned_body(bsrc_hbm, bdl_hbm, nchk_hbm, g_hbm, out_hbm,
                    sidx, didx, dloc, gbuf, zbuf, nvec_buf, acc):
    del dloc
    c = lax.axis_index("c")
    s = lax.axis_index("s")
    _fill(zbuf, 0.0)
    pltpu.sync_copy(nchk_hbm.at[pl.ds(s * 16, 16)], nvec_buf)
    nvec = nvec_buf[pl.ds(0, LANES)]
    for p in range(NBKT // NC):
        bkt = (NBKT // NC) * c + p
        nch0 = _extract(nvec, bkt)
        nch1 = _extract(nvec, 8 + bkt)
        base0 = ((2 * s + 0) * NBKT + bkt) * CAPW
        base1 = ((2 * s + 1) * NBKT + bkt) * CAPW
        _zero_acc(acc, zbuf, s)
        plsc.subcore_barrier()

        @pl.loop(0, nch0 + nch1)
        def _(t):
            addr = jnp.where(t < nch0, base0 + t * CEC,
                             base1 + (t - nch0) * CEC)
            pltpu.sync_copy(bsrc_hbm.at[pl.ds(addr, CEC)], sidx)
            pltpu.sync_copy(bdl_hbm.at[pl.ds(addr, CEC)], didx)
            pltpu.sync_copy(g_hbm.at[sidx], gbuf)
            pltpu.sync_copy(gbuf, acc.at[didx], add=True)

        plsc.subcore_barrier()
        _writeback(acc, out_hbm, bkt, s)
        plsc.subcore_barrier()


def _sc_binned(bsrc, bdl, nchk, g):
    fn = pl.kernel(_sc_binned_body,
                   out_type=jax.ShapeDtypeStruct((N, DH), jnp.float32),
                   mesh=_mesh, scratch_types=_SCRATCH[:5] + [
                       pltpu.VMEM((LANES,), jnp.int32)] + _SCRATCH[5:],
                   compiler_params=_SC_PARAMS)
    return fn(bsrc, bdl, nchk, g)


BLK = 2000
NBLK = N // BLK


def _tc_h0_body(x_ref, w_ref, o_ref):
    o_ref[...] = jnp.dot(x_ref[...], w_ref[...],
                         preferred_element_type=jnp.float32)


def _tc_h0(x, W0):
    return pl.pallas_call(
        _tc_h0_body,
        grid=(NBLK,),
        in_specs=[pl.BlockSpec((BLK, D_IN), lambda i: (i, 0)),
                  pl.BlockSpec((D_IN, DH), lambda i: (0, 0))],
        out_specs=pl.BlockSpec((BLK, DH), lambda i: (i, 0)),
        out_shape=jax.ShapeDtypeStruct((N, DH), jnp.float32),
    )(x, W0)


def _tc_dinv_g0_body(d0_ref, d1_ref, h_ref, dinv_ref, g_ref):
    dinv = lax.rsqrt(d0_ref[...] + d1_ref[...] + 1.0)
    dinv_ref[...] = dinv
    g_ref[...] = dinv * h_ref[...]


def _tc_dinv_g0(d0, d1, h0):
    return pl.pallas_call(
        _tc_dinv_g0_body,
        grid=(NBLK,),
        in_specs=[pl.BlockSpec((BLK, 1), lambda i: (i, 0)),
                  pl.BlockSpec((BLK, 1), lambda i: (i, 0)),
                  pl.BlockSpec((BLK, DH), lambda i: (i, 0))],
        out_specs=[pl.BlockSpec((BLK, 1), lambda i: (i, 0)),
                   pl.BlockSpec((BLK, DH), lambda i: (i, 0))],
        out_shape=[jax.ShapeDtypeStruct((N, 1), jnp.float32),
                   jax.ShapeDtypeStruct((N, DH), jnp.float32)],
    )(d0, d1, h0)


def _tc_layer_body(sp_ref, g_ref, dinv_ref, b_ref, w_ref, o_ref, *, last):
    dinv = dinv_ref[...]
    a = jnp.tanh(dinv * (sp_ref[...] + g_ref[...]) + b_ref[...])
    if last:
        o_ref[...] = a
    else:
        o_ref[...] = dinv * jnp.dot(a, w_ref[...],
                                    preferred_element_type=jnp.float32)


def _tc_layer(sp, g, dinv, b, Wnext, *, last):
    return pl.pallas_call(
        functools.partial(_tc_layer_body, last=last),
        grid=(NBLK,),
        in_specs=[pl.BlockSpec((BLK, DH), lambda i: (i, 0)),
                  pl.BlockSpec((BLK, DH), lambda i: (i, 0)),
                  pl.BlockSpec((BLK, 1), lambda i: (i, 0)),
                  pl.BlockSpec((1, DH), lambda i: (0, 0)),
                  pl.BlockSpec((DH, DH), lambda i: (0, 0))],
        out_specs=pl.BlockSpec((BLK, DH), lambda i: (i, 0)),
        out_shape=jax.ShapeDtypeStruct((N, DH), jnp.float32),
    )(sp, g, dinv, b, Wnext)


def _tc_pool_body(bat_smem, a_ref, bat_ref, hmax_ref, hsum_ref, cnt_ref):
    @pl.when(pl.program_id(0) == 0)
    def _():
        hmax_ref[...] = jnp.full((G, DH), -1e30, jnp.float32)
        hsum_ref[...] = jnp.zeros((G, DH), jnp.float32)
        cnt_ref[...] = jnp.zeros((G, 1), jnp.float32)

    a = a_ref[...]
    bcol = bat_ref[...]
    gmin = bat_smem[0, 0, 0]
    gmax = bat_smem[0, 0, BLK - 1]

    def body(gg, _):
        m = bcol == gg
        bmax = jnp.max(jnp.where(m, a, -1e30), axis=0, keepdims=True)
        bsum = jnp.sum(jnp.where(m, a, 0.0), axis=0, keepdims=True)
        bcnt = jnp.sum(m.astype(jnp.float32), axis=0, keepdims=True)
        hmax_ref[pl.ds(gg, 1), :] = jnp.maximum(hmax_ref[pl.ds(gg, 1), :], bmax)
        hsum_ref[pl.ds(gg, 1), :] = hsum_ref[pl.ds(gg, 1), :] + bsum
        cnt_ref[pl.ds(gg, 1), :] = cnt_ref[pl.ds(gg, 1), :] + bcnt
        return 0

    lax.fori_loop(gmin, gmax + 1, body, 0)


def _tc_pool(a4, batch):
    return pl.pallas_call(
        _tc_pool_body,
        grid=(NBLK,),
        in_specs=[pl.BlockSpec((1, 1, BLK), lambda i: (i, 0, 0),
                               memory_space=pltpu.SMEM),
                  pl.BlockSpec((BLK, DH), lambda i: (i, 0)),
                  pl.BlockSpec((BLK, 1), lambda i: (i, 0))],
        out_specs=[pl.BlockSpec((G, DH), lambda i: (0, 0)),
                   pl.BlockSpec((G, DH), lambda i: (0, 0)),
                   pl.BlockSpec((G, 1), lambda i: (0, 0))],
        out_shape=[jax.ShapeDtypeStruct((G, DH), jnp.float32),
                   jax.ShapeDtypeStruct((G, DH), jnp.float32),
                   jax.ShapeDtypeStruct((G, 1), jnp.float32)],
    )(batch.reshape(NBLK, 1, BLK), a4, batch.reshape(N, 1))


def _tc_final_body(hmax_ref, hsum_ref, cnt_ref, wmax_ref, wmean_ref, bout_ref,
                   o_ref):
    cnt = cnt_ref[...]
    hmax = jnp.where(cnt > 0, hmax_ref[...], 0.0)
    hmean = hsum_ref[...] / jnp.maximum(cnt, 1.0)
    o_ref[...] = (jnp.dot(hmax, wmax_ref[...],
                          preferred_element_type=jnp.float32)
                  + jnp.dot(hmean, wmean_ref[...],
                            preferred_element_type=jnp.float32)
                  + bout_ref[...])


def _tc_final(hmax, hsum, cnt, Wmax, Wmean, bout2d):
    return pl.pallas_call(
        _tc_final_body,
        out_shape=jax.ShapeDtypeStruct((G, 1), jnp.float32),
    )(hmax, hsum, cnt, Wmax, Wmean, bout2d)


def kernel(x, edge_index, batch, W0, b0, W1, b1, W2, b2, W3, b3, Wout, bout):
    bsrc, bdl, nchk, degp = _sc_part(edge_index)
    h0 = _tc_h0(x, W0)
    dinv, g = _tc_dinv_g0(degp[0, :N].reshape(N, 1),
                          degp[1, :N].reshape(N, 1), h0)
    bs = [b0.reshape(1, DH), b1.reshape(1, DH), b2.reshape(1, DH),
          b3.reshape(1, DH)]
    Ws = [W1, W2, W3, W3]
    for l in range(4):
        sp = _sc_binned(bsrc, bdl, nchk, g)
        g = _tc_layer(sp, g, dinv, bs[l], Ws[l], last=(l == 3))
    hmax, hsum, cnt = _tc_pool(g, batch)
    return _tc_final(hmax, hsum, cnt, Wout[:DH], Wout[DH:],
                     bout.reshape(1, 1))
